# baseline (speedup 1.0000x reference)
import functools

import jax
import jax.numpy as jnp
from jax import lax
from jax.experimental import pallas as pl
from jax.experimental.pallas import tpu as pltpu


def _ceil_to(x, m):
    return ((x + m - 1) // m) * m


def _tile_rows(n, cap):
    best = 8
    d = 8
    while d <= min(n, cap):
        if n % d == 0:
            best = d
        d += 8
    return best


def _char_feat_kernel(idx_ref, g_ref, cb_ref, out_ref, *, K, Fp, Lc):
    TN = idx_ref.shape[1]
    NCp = g_ref.shape[0] // K
    P = Lc + K - 1
    ohs = []
    for k in range(K):
        ids = idx_ref[k:k + P, :][:, :, None]
        ohs.append(lax.broadcasted_iota(jnp.int32, (P, TN, NCp), 2) == ids)
    oh = jnp.concatenate(ohs, axis=2).astype(jnp.bfloat16)
    conv = jnp.dot(oh.reshape(P * TN, K * NCp), g_ref[...],
                   preferred_element_type=jnp.float32)
    best = jnp.max(conv.reshape(P, TN, Fp), axis=0)
    out_ref[...] = jnp.tanh(best + cb_ref[...]).astype(out_ref.dtype)


def _bilstm_kernel(word_ref, char_ref, m_ref, win_ref, b_ref, gb_ref, whh_ref,
                   out_ref, zin_ref, h_ref, c_ref, *, NT, Tc, TB, G):
    d = pl.program_id(0)
    j = pl.program_id(1)
    t = lax.rem(j, NT)
    G4 = 4 * G

    @pl.when(t == 0)
    def _():
        h_ref[...] = jnp.zeros_like(h_ref)
        c_ref[...] = jnp.zeros_like(c_ref)

    x = jnp.concatenate([word_ref[...].astype(jnp.bfloat16), char_ref[...]],
                        axis=1)
    zin_ref[...] = (jnp.dot(x, win_ref[0], preferred_element_type=jnp.float32)
                    + b_ref[0] + (1.0 - m_ref[...]) * gb_ref[...])
    whh = whh_ref[0]

    def step(s, carry):
        ls = d * (Tc - 1) + (1 - 2 * d) * s
        r = pl.ds(pl.multiple_of(ls * TB, TB), TB)
        z = zin_ref[r, :] + jnp.dot(h_ref[...], whh,
                                    preferred_element_type=jnp.float32)
        ig = jax.nn.sigmoid(z[:, 0:G])
        fg = jax.nn.sigmoid(z[:, G:2 * G])
        gg = jnp.tanh(z[:, 2 * G:3 * G])
        og = jax.nn.sigmoid(z[:, 3 * G:G4])
        c_new = fg * c_ref[...] + ig * gg
        h_new = og * jnp.tanh(c_new)
        c_ref[...] = c_new
        h_ref[...] = h_new.astype(h_ref.dtype)
        out_ref[r, :] = h_new
        return carry

    lax.fori_loop(0, Tc, step, 0, unroll=8)
    out_ref[...] = out_ref[...] * m_ref[...]


def _pad_gate_lanes(w, H, G):
    if G == H:
        return w
    lead = w.shape[:-1]
    wg = w.reshape(lead + (4, H))
    wg = jnp.pad(wg, [(0, 0)] * len(lead) + [(0, 0), (0, G - H)])
    return wg.reshape(lead + (4 * G,))


def kernel(word_embedd, char_embedd, conv_w, conv_b, wih_f, whh_f, b_f,
           wih_b, whh_b, b_b, input_word, input_char, mask):
    B, T = input_word.shape
    Lc = input_char.shape[2]
    NC, C = char_embedd.shape
    F, _, K = conv_w.shape
    Wd = word_embedd.shape[1]
    H = whh_f.shape[0]

    G = _ceil_to(H, 128)
    Fp = _ceil_to(F, 128)
    NCp = _ceil_to(NC, 128)

    TB = min(128, _ceil_to(B, 8))
    Bp = _ceil_to(B, TB)
    NB = Bp // TB
    Tc = max(1, min(T, 4096 // TB))
    T_pad = _ceil_to(T, Tc)
    NT = T_pad // Tc
    rows_chunk = Tc * TB
    rows_total = NB * NT * rows_chunk

    iw = jnp.pad(input_word, ((0, Bp - B), (0, T_pad - T)))
    iw = jnp.transpose(iw.reshape(NB, TB, T_pad), (0, 2, 1)).reshape(rows_total)
    word_cm = jnp.take(word_embedd, iw, axis=0, mode='clip')

    ic = jnp.pad(input_char, ((0, Bp - B), (0, T_pad - T), (0, 0)))
    ic = jnp.transpose(ic.reshape(NB, TB, T_pad, Lc), (3, 0, 2, 1))
    ic = ic.reshape(Lc, rows_total)
    Lpad = _ceil_to(Lc + 2 * (K - 1), 8)
    ic = jnp.pad(ic, ((K - 1, Lpad - Lc - (K - 1)), (0, 0)),
                 constant_values=-1)

    mask_f32 = mask.astype(jnp.float32)
    m = jnp.pad(mask_f32, ((0, Bp - B), (0, T_pad - T)))
    mask_cm = jnp.transpose(m.reshape(NB, TB, T_pad), (0, 2, 1)).reshape(rows_total, 1)

    w2 = jnp.transpose(conv_w, (1, 2, 0))
    g = jnp.tensordot(char_embedd.astype(jnp.bfloat16).astype(jnp.float32),
                      w2.astype(jnp.bfloat16).astype(jnp.float32),
                      axes=1)
    g = jnp.pad(jnp.transpose(g, (1, 0, 2)),
                ((0, 0), (0, NCp - NC), (0, Fp - F)))
    g = g.reshape(K * NCp, Fp).astype(jnp.bfloat16)
    cb = jnp.pad(conv_b, (0, Fp - F)).reshape(1, Fp).astype(jnp.float32)

    TN = _tile_rows(rows_total, 512)
    char_feat = pl.pallas_call(
        functools.partial(_char_feat_kernel, K=K, Fp=Fp, Lc=Lc),
        out_shape=jax.ShapeDtypeStruct((rows_total, Fp), jnp.bfloat16),
        grid=(rows_total // TN,),
        in_specs=[pl.BlockSpec((Lpad, TN), lambda i: (0, i)),
                  pl.BlockSpec((K * NCp, Fp), lambda i: (0, 0)),
                  pl.BlockSpec((1, Fp), lambda i: (0, 0))],
        out_specs=pl.BlockSpec((TN, Fp), lambda i: (i, 0)),
        compiler_params=pltpu.CompilerParams(
            dimension_semantics=("arbitrary",),
            vmem_limit_bytes=64 * 1024 * 1024),
    )(ic, g, cb)

    def stack_dir(wih, whh_d, b):
        wg = _pad_gate_lanes(wih, H, G)
        wg = jnp.concatenate(
            [wg[:Wd], jnp.pad(wg[Wd:], ((0, Fp - F), (0, 0)))], axis=0)
        hg = _pad_gate_lanes(jnp.pad(whh_d, ((0, G - H), (0, 0))), H, G)
        bg = _pad_gate_lanes(b, H, G)
        return wg, hg, bg

    wf, hf, bf = stack_dir(wih_f, whh_f, b_f)
    wb, hb, bb = stack_dir(wih_b, whh_b, b_b)
    win = jnp.stack([wf, wb]).astype(jnp.bfloat16)
    whh = jnp.stack([hf, hb]).astype(jnp.bfloat16)
    bst = jnp.stack([bf, bb]).astype(jnp.float32)
    BIG = jnp.float32(1e9)
    gbias = jnp.concatenate([jnp.full((1, G), -BIG, jnp.float32),
                             jnp.full((1, G), BIG, jnp.float32),
                             jnp.zeros((1, 2 * G), jnp.float32)], axis=1)

    def chunk_idx(d, j):
        nb = j // NT
        t = j % NT
        return nb * NT + d * (NT - 1) + (1 - 2 * d) * t

    in_idx = lambda d, j: (chunk_idx(d, j), 0)
    wgt_idx = lambda d, j: (d, 0, 0)
    out_idx = lambda d, j: (chunk_idx(d, j), d)

    out = pl.pallas_call(
        functools.partial(_bilstm_kernel, NT=NT, Tc=Tc, TB=TB, G=G),
        out_shape=jax.ShapeDtypeStruct((rows_total, 2 * G), jnp.float32),
        grid=(2, NB * NT),
        in_specs=[pl.BlockSpec((rows_chunk, Wd), in_idx),
                  pl.BlockSpec((rows_chunk, Fp), in_idx),
                  pl.BlockSpec((rows_chunk, 1), in_idx),
                  pl.BlockSpec((1, Wd + Fp, 4 * G), wgt_idx),
                  pl.BlockSpec((1, 1, 4 * G), wgt_idx),
                  pl.BlockSpec((1, 4 * G), lambda d, j: (0, 0)),
                  pl.BlockSpec((1, G, 4 * G), wgt_idx)],
        out_specs=pl.BlockSpec((rows_chunk, G), out_idx),
        scratch_shapes=[pltpu.VMEM((rows_chunk, 4 * G), jnp.float32),
                        pltpu.VMEM((TB, G), jnp.bfloat16),
                        pltpu.VMEM((TB, G), jnp.float32)],
        compiler_params=pltpu.CompilerParams(
            dimension_semantics=("arbitrary", "arbitrary"),
            vmem_limit_bytes=64 * 1024 * 1024),
    )(word_cm, char_feat, mask_cm, win, bst, gbias, whh)

    y = out.reshape(NB, NT * Tc, TB, 2 * G)
    y = jnp.transpose(y, (0, 2, 1, 3)).reshape(Bp, T_pad, 2 * G)
    if G == H:
        output = y[:B, :T, :]
        lm_fw = output[:, :, :H]
        lm_bw = output[:, :, H:]
    else:
        lm_fw = y[:B, :T, :H]
        lm_bw = y[:B, :T, G:G + H]
        output = jnp.concatenate([lm_fw, lm_bw], axis=-1)
    length = mask_f32.sum(axis=1).astype(jnp.int32)
    return output, mask, length, lm_fw, lm_bw


# speedup vs baseline: 1.2799x; 1.2799x over previous
import functools

import jax
import jax.numpy as jnp
from jax import lax
from jax.experimental import pallas as pl
from jax.experimental.pallas import tpu as pltpu


def _ceil_to(x, m):
    return ((x + m - 1) // m) * m


def _tile_rows(n, cap):
    best = 8
    d = 8
    while d <= min(n, cap):
        if n % d == 0:
            best = d
        d += 8
    return best


def _char_feat_kernel(idx_ref, g_ref, cb_ref, out_ref, *, K, Fp, Lc):
    TN = idx_ref.shape[1]
    NCp = g_ref.shape[0]
    ids = idx_ref[...][:, :, None]
    oh = (lax.broadcasted_iota(jnp.int32, (Lc, TN, NCp), 2) == ids)
    a = jnp.dot(oh.astype(jnp.bfloat16).reshape(Lc * TN, NCp), g_ref[...],
                preferred_element_type=jnp.float32)
    a = a.astype(jnp.bfloat16).reshape(Lc, TN, K * Fp)
    taps = [a[:, :, k * Fp:(k + 1) * Fp] for k in range(K)]

    best = None
    for l in range(Lc + K - 1):
        s = None
        for k in range(K):
            m = l - (K - 1) + k
            if 0 <= m < Lc:
                s = taps[k][m] if s is None else s + taps[k][m]
        best = s if best is None else jnp.maximum(best, s)
    out_ref[...] = jnp.tanh(best.astype(jnp.float32)
                            + cb_ref[...]).astype(out_ref.dtype)


def _bilstm_kernel(word_ref, char_ref, m_ref, win_ref, b_ref, gb_ref, whh_ref,
                   out_ref, zin_ref, h_ref, c_ref, *, NT, Tc, TB, G):
    d = pl.program_id(0)
    j = pl.program_id(1)
    t = lax.rem(j, NT)
    G4 = 4 * G

    @pl.when(t == 0)
    def _():
        h_ref[...] = jnp.zeros_like(h_ref)
        c_ref[...] = jnp.zeros_like(c_ref)

    x = jnp.concatenate([word_ref[...].astype(jnp.bfloat16), char_ref[...]],
                        axis=1)
    zin_ref[...] = (jnp.dot(x, win_ref[0], preferred_element_type=jnp.float32)
                    + b_ref[0] + (1.0 - m_ref[...]) * gb_ref[...])
    whh = whh_ref[0]

    def step(s, carry):
        ls = d * (Tc - 1) + (1 - 2 * d) * s
        r = pl.ds(pl.multiple_of(ls * TB, TB), TB)
        h = h_ref[...]
        z1 = zin_ref[r, 0:2 * G] + jnp.dot(h, whh[:, 0:2 * G],
                                           preferred_element_type=jnp.float32)
        z2 = zin_ref[r, 2 * G:G4] + jnp.dot(h, whh[:, 2 * G:G4],
                                            preferred_element_type=jnp.float32)
        ig = jax.nn.sigmoid(z1[:, 0:G])
        fg = jax.nn.sigmoid(z1[:, G:2 * G])
        gg = jnp.tanh(z2[:, 0:G])
        og = jax.nn.sigmoid(z2[:, G:2 * G])
        c_new = fg * c_ref[...] + ig * gg
        h_new = og * jnp.tanh(c_new)
        c_ref[...] = c_new
        h_ref[...] = h_new.astype(h_ref.dtype)
        out_ref[r, :] = h_new
        return carry

    lax.fori_loop(0, Tc, step, 0, unroll=8)
    out_ref[...] = out_ref[...] * m_ref[...]


def _pad_gate_lanes(w, H, G):
    if G == H:
        return w
    lead = w.shape[:-1]
    wg = w.reshape(lead + (4, H))
    wg = jnp.pad(wg, [(0, 0)] * len(lead) + [(0, 0), (0, G - H)])
    return wg.reshape(lead + (4 * G,))


def kernel(word_embedd, char_embedd, conv_w, conv_b, wih_f, whh_f, b_f,
           wih_b, whh_b, b_b, input_word, input_char, mask):
    B, T = input_word.shape
    Lc = input_char.shape[2]
    NC, C = char_embedd.shape
    F, _, K = conv_w.shape
    Wd = word_embedd.shape[1]
    H = whh_f.shape[0]

    G = _ceil_to(H, 128)
    Fp = _ceil_to(F, 128)
    NCp = _ceil_to(NC, 128)

    TB = min(128, _ceil_to(B, 8))
    Bp = _ceil_to(B, TB)
    NB = Bp // TB
    Tc = max(1, min(T, 4096 // TB))
    T_pad = _ceil_to(T, Tc)
    NT = T_pad // Tc
    rows_chunk = Tc * TB
    rows_total = NB * NT * rows_chunk

    iw = jnp.pad(input_word, ((0, Bp - B), (0, T_pad - T)))
    iw = jnp.transpose(iw.reshape(NB, TB, T_pad), (0, 2, 1)).reshape(rows_total)
    word_cm = jnp.take(word_embedd, iw, axis=0, mode='clip')

    ic = jnp.pad(input_char, ((0, Bp - B), (0, T_pad - T), (0, 0)))
    ic = jnp.transpose(ic.reshape(NB, TB, T_pad, Lc), (3, 0, 2, 1))
    ic = ic.reshape(Lc, rows_total)

    mask_f32 = mask.astype(jnp.float32)
    m = jnp.pad(mask_f32, ((0, Bp - B), (0, T_pad - T)))
    mask_cm = jnp.transpose(m.reshape(NB, TB, T_pad), (0, 2, 1)).reshape(rows_total, 1)

    w2 = jnp.transpose(conv_w, (1, 2, 0))
    w2 = jnp.pad(w2, ((0, 0), (0, 0), (0, Fp - F))).reshape(C, K * Fp)
    g = jnp.dot(char_embedd.astype(jnp.bfloat16).astype(jnp.float32),
                w2.astype(jnp.bfloat16).astype(jnp.float32))
    g = jnp.pad(g, ((0, NCp - NC), (0, 0))).astype(jnp.bfloat16)
    cb = jnp.pad(conv_b, (0, Fp - F)).reshape(1, Fp).astype(jnp.float32)

    TN = _tile_rows(rows_total, 512)
    char_feat = pl.pallas_call(
        functools.partial(_char_feat_kernel, K=K, Fp=Fp, Lc=Lc),
        out_shape=jax.ShapeDtypeStruct((rows_total, Fp), jnp.bfloat16),
        grid=(rows_total // TN,),
        in_specs=[pl.BlockSpec((Lc, TN), lambda i: (0, i)),
                  pl.BlockSpec((NCp, K * Fp), lambda i: (0, 0)),
                  pl.BlockSpec((1, Fp), lambda i: (0, 0))],
        out_specs=pl.BlockSpec((TN, Fp), lambda i: (i, 0)),
        compiler_params=pltpu.CompilerParams(
            dimension_semantics=("arbitrary",),
            vmem_limit_bytes=64 * 1024 * 1024),
    )(ic, g, cb)

    def stack_dir(wih, whh_d, b):
        wg = _pad_gate_lanes(wih, H, G)
        wg = jnp.concatenate(
            [wg[:Wd], jnp.pad(wg[Wd:], ((0, Fp - F), (0, 0)))], axis=0)
        hg = _pad_gate_lanes(jnp.pad(whh_d, ((0, G - H), (0, 0))), H, G)
        bg = _pad_gate_lanes(b, H, G)
        return wg, hg, bg

    wf, hf, bf = stack_dir(wih_f, whh_f, b_f)
    wb, hb, bb = stack_dir(wih_b, whh_b, b_b)
    win = jnp.stack([wf, wb]).astype(jnp.bfloat16)
    whh = jnp.stack([hf, hb]).astype(jnp.bfloat16)
    bst = jnp.stack([bf, bb]).astype(jnp.float32)
    BIG = jnp.float32(1e9)
    gbias = jnp.concatenate([jnp.full((1, G), -BIG, jnp.float32),
                             jnp.full((1, G), BIG, jnp.float32),
                             jnp.zeros((1, 2 * G), jnp.float32)], axis=1)

    def chunk_idx(d, j):
        nb = j // NT
        t = j % NT
        return nb * NT + d * (NT - 1) + (1 - 2 * d) * t

    in_idx = lambda d, j: (chunk_idx(d, j), 0)
    wgt_idx = lambda d, j: (d, 0, 0)
    out_idx = lambda d, j: (chunk_idx(d, j), d)

    out = pl.pallas_call(
        functools.partial(_bilstm_kernel, NT=NT, Tc=Tc, TB=TB, G=G),
        out_shape=jax.ShapeDtypeStruct((rows_total, 2 * G), jnp.float32),
        grid=(2, NB * NT),
        in_specs=[pl.BlockSpec((rows_chunk, Wd), in_idx),
                  pl.BlockSpec((rows_chunk, Fp), in_idx),
                  pl.BlockSpec((rows_chunk, 1), in_idx),
                  pl.BlockSpec((1, Wd + Fp, 4 * G), wgt_idx),
                  pl.BlockSpec((1, 1, 4 * G), wgt_idx),
                  pl.BlockSpec((1, 4 * G), lambda d, j: (0, 0)),
                  pl.BlockSpec((1, G, 4 * G), wgt_idx)],
        out_specs=pl.BlockSpec((rows_chunk, G), out_idx),
        scratch_shapes=[pltpu.VMEM((rows_chunk, 4 * G), jnp.float32),
                        pltpu.VMEM((TB, G), jnp.bfloat16),
                        pltpu.VMEM((TB, G), jnp.float32)],
        compiler_params=pltpu.CompilerParams(
            dimension_semantics=("arbitrary", "arbitrary"),
            vmem_limit_bytes=64 * 1024 * 1024),
    )(word_cm, char_feat, mask_cm, win, bst, gbias, whh)

    y = out.reshape(NB, NT * Tc, TB, 2 * G)
    y = jnp.transpose(y, (0, 2, 1, 3)).reshape(Bp, T_pad, 2 * G)
    if G == H:
        output = y[:B, :T, :]
        lm_fw = output[:, :, :H]
        lm_bw = output[:, :, H:]
    else:
        lm_fw = y[:B, :T, :H]
        lm_bw = y[:B, :T, G:G + H]
        output = jnp.concatenate([lm_fw, lm_bw], axis=-1)
    length = mask_f32.sum(axis=1).astype(jnp.int32)
    return output, mask, length, lm_fw, lm_bw


# speedup vs baseline: 1.3191x; 1.0307x over previous
import functools

import jax
import jax.numpy as jnp
from jax import lax
from jax.experimental import pallas as pl
from jax.experimental.pallas import tpu as pltpu


def _ceil_to(x, m):
    return ((x + m - 1) // m) * m


def _tile_rows(n, cap):
    best = 8
    d = 8
    while d <= min(n, cap):
        if n % d == 0:
            best = d
        d += 8
    return best


def _char_feat_kernel(idx_ref, g_ref, cb_ref, out_ref, *, K, Fp, Lc):
    TN = idx_ref.shape[1]
    NCp = g_ref.shape[0]
    ids = idx_ref[...][:, :, None]
    oh = (lax.broadcasted_iota(jnp.int32, (Lc, TN, NCp), 2) == ids)
    a = jnp.dot(oh.astype(jnp.bfloat16).reshape(Lc * TN, NCp), g_ref[...],
                preferred_element_type=jnp.float32)
    a = a.astype(jnp.bfloat16).reshape(Lc, TN, K * Fp)
    taps = [a[:, :, k * Fp:(k + 1) * Fp] for k in range(K)]

    best = None
    for l in range(Lc + K - 1):
        s = None
        for k in range(K):
            m = l - (K - 1) + k
            if 0 <= m < Lc:
                s = taps[k][m] if s is None else s + taps[k][m]
        best = s if best is None else jnp.maximum(best, s)
    out_ref[...] = jnp.tanh(best.astype(jnp.float32)
                            + cb_ref[...]).astype(out_ref.dtype)


def _bilstm_kernel(word_ref, char_ref, m_ref, w_ref, b_ref, gb_ref,
                   out_ref, zout_ref, h_ref, c_ref, *, NT, Tc, TB, G):
    d = pl.program_id(0)
    j = pl.program_id(1)
    t = lax.rem(j, NT)
    G4 = 4 * G

    @pl.when(t == 0)
    def _():
        h_ref[...] = jnp.zeros_like(h_ref)
        c_ref[...] = jnp.zeros_like(c_ref)

    w = w_ref[0]
    bias = b_ref[0]
    gb = gb_ref[...]

    IN = word_ref.shape[1] + char_ref.shape[1]

    def step(s, carry):
        ls = d * (Tc - 1) + (1 - 2 * d) * s
        r = pl.ds(pl.multiple_of(ls * TB, TB), TB)
        x = jnp.concatenate([word_ref[r, :].astype(jnp.bfloat16),
                             char_ref[r, :]], axis=1)
        mb = bias + (1.0 - m_ref[r, :]) * gb
        h = h_ref[...]
        zx1 = mb[:, 0:2 * G] + jnp.dot(x, w[0:IN, 0:2 * G],
                                       preferred_element_type=jnp.float32)
        zx2 = mb[:, 2 * G:G4] + jnp.dot(x, w[0:IN, 2 * G:G4],
                                        preferred_element_type=jnp.float32)
        z1 = zx1 + jnp.dot(h, w[IN:, 0:2 * G],
                           preferred_element_type=jnp.float32)
        z2 = zx2 + jnp.dot(h, w[IN:, 2 * G:G4],
                           preferred_element_type=jnp.float32)
        ig = jax.nn.sigmoid(z1[:, 0:G])
        fg = jax.nn.sigmoid(z1[:, G:2 * G])
        gg = jnp.tanh(z2[:, 0:G])
        og = jax.nn.sigmoid(z2[:, G:2 * G])
        c_new = fg * c_ref[...] + ig * gg
        h_new = og * jnp.tanh(c_new)
        c_ref[...] = c_new
        h_ref[...] = h_new.astype(h_ref.dtype)
        zout_ref[r, :] = h_new
        return carry

    lax.fori_loop(0, Tc, step, 0, unroll=8)
    y = zout_ref[...] * m_ref[...]
    out_ref[...] = jnp.transpose(y.reshape(Tc, TB, G), (1, 0, 2))


def _pad_gate_lanes(w, H, G):
    if G == H:
        return w
    lead = w.shape[:-1]
    wg = w.reshape(lead + (4, H))
    wg = jnp.pad(wg, [(0, 0)] * len(lead) + [(0, 0), (0, G - H)])
    return wg.reshape(lead + (4 * G,))


def kernel(word_embedd, char_embedd, conv_w, conv_b, wih_f, whh_f, b_f,
           wih_b, whh_b, b_b, input_word, input_char, mask):
    B, T = input_word.shape
    Lc = input_char.shape[2]
    NC, C = char_embedd.shape
    F, _, K = conv_w.shape
    Wd = word_embedd.shape[1]
    H = whh_f.shape[0]

    G = _ceil_to(H, 128)
    Fp = _ceil_to(F, 128)
    NCp = _ceil_to(NC, 128)

    TB = min(128, _ceil_to(B, 8))
    Bp = _ceil_to(B, TB)
    NB = Bp // TB
    Tc = max(1, min(T, 4096 // TB))
    T_pad = _ceil_to(T, Tc)
    NT = T_pad // Tc
    rows_chunk = Tc * TB
    rows_total = NB * NT * rows_chunk

    iw = jnp.pad(input_word, ((0, Bp - B), (0, T_pad - T)))
    iw = jnp.transpose(iw.reshape(NB, TB, T_pad), (0, 2, 1)).reshape(rows_total)
    word_cm = jnp.take(word_embedd, iw, axis=0, mode='clip')

    ic = jnp.pad(input_char, ((0, Bp - B), (0, T_pad - T), (0, 0)))
    ic = jnp.transpose(ic.reshape(NB, TB, T_pad, Lc), (3, 0, 2, 1))
    ic = ic.reshape(Lc, rows_total)

    mask_f32 = mask.astype(jnp.float32)
    m = jnp.pad(mask_f32, ((0, Bp - B), (0, T_pad - T)))
    mask_cm = jnp.transpose(m.reshape(NB, TB, T_pad), (0, 2, 1)).reshape(rows_total, 1)

    w2 = jnp.transpose(conv_w, (1, 2, 0))
    w2 = jnp.pad(w2, ((0, 0), (0, 0), (0, Fp - F))).reshape(C, K * Fp)
    g = jnp.dot(char_embedd.astype(jnp.bfloat16).astype(jnp.float32),
                w2.astype(jnp.bfloat16).astype(jnp.float32))
    g = jnp.pad(g, ((0, NCp - NC), (0, 0))).astype(jnp.bfloat16)
    cb = jnp.pad(conv_b, (0, Fp - F)).reshape(1, Fp).astype(jnp.float32)

    TN = _tile_rows(rows_total, 512)
    char_feat = pl.pallas_call(
        functools.partial(_char_feat_kernel, K=K, Fp=Fp, Lc=Lc),
        out_shape=jax.ShapeDtypeStruct((rows_total, Fp), jnp.bfloat16),
        grid=(rows_total // TN,),
        in_specs=[pl.BlockSpec((Lc, TN), lambda i: (0, i)),
                  pl.BlockSpec((NCp, K * Fp), lambda i: (0, 0)),
                  pl.BlockSpec((1, Fp), lambda i: (0, 0))],
        out_specs=pl.BlockSpec((TN, Fp), lambda i: (i, 0)),
        compiler_params=pltpu.CompilerParams(
            dimension_semantics=("arbitrary",),
            vmem_limit_bytes=64 * 1024 * 1024),
    )(ic, g, cb)

    def stack_dir(wih, whh_d, b):
        wg = _pad_gate_lanes(wih, H, G)
        hg = _pad_gate_lanes(jnp.pad(whh_d, ((0, G - H), (0, 0))), H, G)
        wg = jnp.concatenate(
            [wg[:Wd], jnp.pad(wg[Wd:], ((0, Fp - F), (0, 0))), hg], axis=0)
        return wg, _pad_gate_lanes(b, H, G)

    wf, bf = stack_dir(wih_f, whh_f, b_f)
    wb, bb = stack_dir(wih_b, whh_b, b_b)
    wst = jnp.stack([wf, wb]).astype(jnp.bfloat16)
    bst = jnp.stack([bf, bb]).astype(jnp.float32)
    BIG = jnp.float32(1e9)
    gbias = jnp.concatenate([jnp.full((1, G), -BIG, jnp.float32),
                             jnp.full((1, G), BIG, jnp.float32),
                             jnp.zeros((1, 2 * G), jnp.float32)], axis=1)

    def chunk_idx(d, j):
        nb = j // NT
        t = j % NT
        return nb * NT + d * (NT - 1) + (1 - 2 * d) * t

    in_idx = lambda d, j: (chunk_idx(d, j), 0)
    wgt_idx = lambda d, j: (d, 0, 0)
    out_idx = lambda d, j: (j // NT, chunk_idx(d, j) % NT, d)

    y = pl.pallas_call(
        functools.partial(_bilstm_kernel, NT=NT, Tc=Tc, TB=TB, G=G),
        out_shape=jax.ShapeDtypeStruct((Bp, T_pad, 2 * G), jnp.float32),
        grid=(2, NB * NT),
        in_specs=[pl.BlockSpec((rows_chunk, Wd), in_idx),
                  pl.BlockSpec((rows_chunk, Fp), in_idx),
                  pl.BlockSpec((rows_chunk, 1), in_idx),
                  pl.BlockSpec((1, Wd + Fp + G, 4 * G), wgt_idx),
                  pl.BlockSpec((1, 1, 4 * G), wgt_idx),
                  pl.BlockSpec((1, 4 * G), lambda d, j: (0, 0))],
        out_specs=pl.BlockSpec((TB, Tc, G), out_idx),
        scratch_shapes=[pltpu.VMEM((rows_chunk, G), jnp.float32),
                        pltpu.VMEM((TB, G), jnp.bfloat16),
                        pltpu.VMEM((TB, G), jnp.float32)],
        compiler_params=pltpu.CompilerParams(
            dimension_semantics=("arbitrary", "arbitrary"),
            vmem_limit_bytes=64 * 1024 * 1024),
    )(word_cm, char_feat, mask_cm, wst, bst, gbias)

    if G == H:
        output = y[:B, :T, :]
        lm_fw = output[:, :, :H]
        lm_bw = output[:, :, H:]
    else:
        lm_fw = y[:B, :T, :H]
        lm_bw = y[:B, :T, G:G + H]
        output = jnp.concatenate([lm_fw, lm_bw], axis=-1)
    length = mask_f32.sum(axis=1).astype(jnp.int32)
    return output, mask, length, lm_fw, lm_bw
